# baseline (speedup 1.0000x reference)
import jax
import jax.numpy as jnp
import numpy as np
from jax import lax
from jax.experimental import pallas as pl
from jax.experimental.pallas import tpu as pltpu
from jax.experimental.pallas import tpu_sc as plsc

_NUM_CLASSES = 14161
_NUM_CHANNELS = 32
_GRID_DIM = 10
_Z = 160000

_NW = 32
_LANES = 16
_CHUNK = 256
_NCHUNKS = _Z // _CHUNK
_MAXJ = -(-_NCHUNKS // _NW)
_SUB = 128
_NSUB = 3 * _CHUNK // _SUB
_TILE_ROWS = _NUM_CHANNELS // 8
_TILE_COLS = _Z // 128

_COS_COEF = (
    0.99999999, -19.7392045, 64.93911746, -85.45013953,
    60.16763095, -25.96759925, 6.52865816,
)


def _basis(xv, sf):
  lo = (sf - 2.0) * 0.1
  up = (sf + 1.0) * 0.1
  a = jnp.minimum(jnp.maximum(xv, lo), up)
  v = (a - lo) * jnp.float32(10.0 / 3.0) - 0.5
  wq = v * v
  c2 = jnp.float32(_COS_COEF[6])
  for k in (5, 4, 3, 2, 1, 0):
    c2 = c2 * wq + jnp.float32(_COS_COEF[k])
  h = 0.5 + 0.5 * c2
  return h * h


def _sc_body(x_hbm, cls_hbm, tab_hbm, out_hbm,
             x_c, cls_c, ibuf, bbuf, wbuf, obuf, gsem, osem):
  wid = lax.axis_index("s") * 2 + lax.axis_index("c")

  iota = lax.iota(jnp.int32, _LANES)

  def chunk_id(j):
    return wid + _NW * j

  def prep_and_fire(j):
    buf = j % 2
    cid = chunk_id(j)

    @pl.when(cid < _NCHUNKS)
    def _():
      base = cid * _CHUNK
      pltpu.sync_copy(x_hbm.at[pl.ds(base, _CHUNK)], x_c.at[buf])
      pltpu.sync_copy(cls_hbm.at[pl.ds(base, _CHUNK)], cls_c.at[buf])

      @pl.loop(0, _CHUNK // _LANES)
      def _pass1(g):
        off = g * _LANES
        xv = x_c[buf, pl.ds(off, _LANES)]
        cv = cls_c[buf, pl.ds(off, _LANES)]
        t10 = xv * 10.0
        swi = jnp.minimum(jnp.maximum(t10.astype(jnp.int32), 0), 7)
        row0 = cv * 10 + swi
        sf = swi.astype(jnp.float32)
        for k in range(3):
          ibuf[buf, pl.ds(k * _CHUNK + off, _LANES)] = row0 + k
          bbuf[buf, pl.ds(k * _CHUNK + off, _LANES)] = _basis(
              xv, sf + jnp.float32(k))

      for t in range(_NSUB):
        idx = ibuf.at[buf, pl.ds(t * _SUB, _SUB)]
        pltpu.async_copy(tab_hbm.at[idx],
                         wbuf.at[buf, pl.ds(t * _SUB, _SUB)], gsem.at[buf])

  def wait_gathers(j):
    buf = j % 2

    @pl.when(chunk_id(j) < _NCHUNKS)
    def _():
      for t in range(_NSUB):
        pltpu.make_async_copy(tab_hbm.at[pl.ds(0, _SUB)],
                              wbuf.at[buf, pl.ds(t * _SUB, _SUB)],
                              gsem.at[buf]).wait()

  def wait_out(j):
    buf = j % 2

    @pl.when(chunk_id(j) < _NCHUNKS)
    def _():
      pltpu.make_async_copy(out_hbm.at[pl.ds(0, 8 * 1024)], obuf.at[buf],
                            osem.at[buf]).wait()

  def compute_and_store(j):
    buf = j % 2
    cid = chunk_id(j)

    @pl.when(cid < _NCHUNKS)
    def _():
      @plsc.parallel_loop(0, _CHUNK // _LANES, unroll=1)
      def _body(eg):
        e0 = eg * _LANES
        tcl = eg // 8
        esub = (eg % 8) * _LANES
        b0 = bbuf[buf, pl.ds(e0, _LANES)]
        b1 = bbuf[buf, pl.ds(_CHUNK + e0, _LANES)]
        b2 = bbuf[buf, pl.ds(2 * _CHUNK + e0, _LANES)]
        r0 = iota + e0
        r1 = r0 + _CHUNK
        r2 = r0 + 2 * _CHUNK

        @plsc.parallel_loop(0, 4, unroll=1)
        def _octet(co):
          dbase = (co * 2 + tcl) * 1024 + esub
          for cc in range(8):
            cv = jnp.full((_LANES,), co * 8 + cc, jnp.int32)
            g0 = plsc.load_gather(wbuf.at[buf], [r0, cv])
            g1 = plsc.load_gather(wbuf.at[buf], [r1, cv])
            g2 = plsc.load_gather(wbuf.at[buf], [r2, cv])
            acc = g0 * b0 + g1 * b1 + g2 * b2
            obuf[buf, pl.ds(dbase + cc * 128, _LANES)] = acc

      for b in range(8):
        tr, tcl = b // 2, b % 2
        off = (tr * _TILE_COLS + 2 * cid + tcl) * 1024
        pltpu.async_copy(obuf.at[buf, pl.ds(b * 1024, 1024)],
                         out_hbm.at[pl.ds(off, 1024)], osem.at[buf])

  prep_and_fire(0)
  for j in range(_MAXJ):
    if j + 1 < _MAXJ:
      prep_and_fire(j + 1)
    wait_gathers(j)
    if j >= 2:
      wait_out(j - 2)
    compute_and_store(j)
  wait_out(_MAXJ - 2)
  wait_out(_MAXJ - 1)


@jax.jit
def _run(xf, cls, tab2):
  mesh = plsc.VectorSubcoreMesh(core_axis_name="c", subcore_axis_name="s")
  f = pl.kernel(
      _sc_body,
      out_type=jax.ShapeDtypeStruct((_Z * _NUM_CHANNELS,), jnp.float32),
      mesh=mesh,
      scratch_types=[
          pltpu.VMEM((2, _CHUNK), jnp.float32),
          pltpu.VMEM((2, _CHUNK), jnp.int32),
          pltpu.VMEM((2, 3 * _CHUNK), jnp.int32),
          pltpu.VMEM((2, 3 * _CHUNK), jnp.float32),
          pltpu.VMEM((2, 3 * _CHUNK, _NUM_CHANNELS), jnp.float32),
          pltpu.VMEM((2, 8 * 1024), jnp.float32),
          pltpu.SemaphoreType.DMA((2,)),
          pltpu.SemaphoreType.DMA((2,)),
      ],
      compiler_params=pltpu.CompilerParams(use_tc_tiling_on_sc=False,
                                           needs_layout_passes=False),
  )
  return f(xf, cls, tab2)


def kernel(x, classes, table):
  tab2 = table.reshape(_NUM_CLASSES, _NUM_CHANNELS, _GRID_DIM).transpose(
      0, 2, 1).reshape(_NUM_CLASSES * _GRID_DIM, _NUM_CHANNELS)
  xf = x.reshape(-1)
  cls = classes.astype(jnp.int32)
  flat = _run(xf, cls, tab2)
  out = flat.reshape(_TILE_ROWS, _TILE_COLS, 8, 128).transpose(
      0, 2, 1, 3).reshape(_NUM_CHANNELS, _Z).T
  return out


# speedup vs baseline: 1.6429x; 1.6429x over previous
import jax
import jax.numpy as jnp
import numpy as np
from jax import lax
from jax.experimental import pallas as pl
from jax.experimental.pallas import tpu as pltpu
from jax.experimental.pallas import tpu_sc as plsc

_NUM_CLASSES = 14161
_NUM_CHANNELS = 32
_GRID_DIM = 10
_Z = 160000

_NW = 32
_LANES = 16
_CHUNK = 256
_NCHUNKS = _Z // _CHUNK
_MAXJ = -(-_NCHUNKS // _NW)
_SUB = 128
_NSUB = 3 * _CHUNK // _SUB
_COS_COEF = (
    0.99999999, -19.7392045, 64.93911746, -85.45013953,
    60.16763095, -25.96759925, 6.52865816,
)


def _basis(xv, sf):
  lo = (sf - 2.0) * 0.1
  up = (sf + 1.0) * 0.1
  a = jnp.minimum(jnp.maximum(xv, lo), up)
  v = (a - lo) * jnp.float32(10.0 / 3.0) - 0.5
  wq = v * v
  c2 = jnp.float32(_COS_COEF[6])
  for k in (5, 4, 3, 2, 1, 0):
    c2 = c2 * wq + jnp.float32(_COS_COEF[k])
  h = 0.5 + 0.5 * c2
  return h * h


def _sc_body(x_hbm, cls_hbm, tab_hbm, out_hbm,
             x_c, cls_c, ibuf, bbuf, wbuf, obuf, gsem, osem):
  wid = lax.axis_index("s") * 2 + lax.axis_index("c")

  def chunk_id(j):
    return wid + _NW * j

  def prep_and_fire(j):
    buf = j % 2
    cid = chunk_id(j)

    @pl.when(cid < _NCHUNKS)
    def _():
      base = cid * _CHUNK
      pltpu.sync_copy(x_hbm.at[pl.ds(base, _CHUNK)], x_c.at[buf])
      pltpu.sync_copy(cls_hbm.at[pl.ds(base, _CHUNK)], cls_c.at[buf])

      @pl.loop(0, _CHUNK // _LANES)
      def _pass1(g):
        off = g * _LANES
        xv = x_c[buf, pl.ds(off, _LANES)]
        cv = cls_c[buf, pl.ds(off, _LANES)]
        t10 = xv * 10.0
        swi = jnp.minimum(jnp.maximum(t10.astype(jnp.int32), 0), 7)
        row0 = cv * 10 + swi
        sf = swi.astype(jnp.float32)
        for k in range(3):
          ibuf[buf, pl.ds(k * _CHUNK + off, _LANES)] = row0 + k
          bbuf[buf, pl.ds(k * _CHUNK + off, _LANES)] = _basis(
              xv, sf + jnp.float32(k))

      for t in range(_NSUB):
        idx = ibuf.at[buf, pl.ds(t * _SUB, _SUB)]
        pltpu.async_copy(tab_hbm.at[idx],
                         wbuf.at[buf, pl.ds(t * _SUB, _SUB)], gsem.at[buf])

  def wait_gathers(j):
    buf = j % 2

    @pl.when(chunk_id(j) < _NCHUNKS)
    def _():
      for t in range(_NSUB):
        pltpu.make_async_copy(tab_hbm.at[pl.ds(0, _SUB)],
                              wbuf.at[buf, pl.ds(t * _SUB, _SUB)],
                              gsem.at[buf]).wait()

  def wait_out(j):
    buf = j % 2

    @pl.when(chunk_id(j) < _NCHUNKS)
    def _():
      pltpu.make_async_copy(out_hbm.at[pl.ds(0, _CHUNK)], obuf.at[buf],
                            osem.at[buf]).wait()

  def compute_and_store(j):
    buf = j % 2
    cid = chunk_id(j)

    @pl.when(cid < _NCHUNKS)
    def _():
      @plsc.parallel_loop(0, _CHUNK, unroll=2)
      def _body(i):
        b0 = bbuf[buf, pl.ds(i, _LANES)][0]
        b1 = bbuf[buf, pl.ds(_CHUNK + i, _LANES)][0]
        b2 = bbuf[buf, pl.ds(2 * _CHUNK + i, _LANES)][0]
        w00 = wbuf[buf, i, pl.ds(0, _LANES)]
        w01 = wbuf[buf, i, pl.ds(_LANES, _LANES)]
        w10 = wbuf[buf, _CHUNK + i, pl.ds(0, _LANES)]
        w11 = wbuf[buf, _CHUNK + i, pl.ds(_LANES, _LANES)]
        w20 = wbuf[buf, 2 * _CHUNK + i, pl.ds(0, _LANES)]
        w21 = wbuf[buf, 2 * _CHUNK + i, pl.ds(_LANES, _LANES)]
        obuf[buf, i, pl.ds(0, _LANES)] = w00 * b0 + w10 * b1 + w20 * b2
        obuf[buf, i, pl.ds(_LANES, _LANES)] = w01 * b0 + w11 * b1 + w21 * b2

      pltpu.async_copy(obuf.at[buf], out_hbm.at[pl.ds(cid * _CHUNK, _CHUNK)],
                       osem.at[buf])

  prep_and_fire(0)
  for j in range(_MAXJ):
    if j + 1 < _MAXJ:
      prep_and_fire(j + 1)
    wait_gathers(j)
    if j >= 2:
      wait_out(j - 2)
    compute_and_store(j)
  wait_out(_MAXJ - 2)
  wait_out(_MAXJ - 1)


@jax.jit
def _run(xf, cls, tab2):
  mesh = plsc.VectorSubcoreMesh(core_axis_name="c", subcore_axis_name="s")
  f = pl.kernel(
      _sc_body,
      out_type=jax.ShapeDtypeStruct((_Z, _NUM_CHANNELS), jnp.float32),
      mesh=mesh,
      scratch_types=[
          pltpu.VMEM((2, _CHUNK), jnp.float32),
          pltpu.VMEM((2, _CHUNK), jnp.int32),
          pltpu.VMEM((2, 3 * _CHUNK), jnp.int32),
          pltpu.VMEM((2, 3 * _CHUNK + _LANES), jnp.float32),
          pltpu.VMEM((2, 3 * _CHUNK, _NUM_CHANNELS), jnp.float32),
          pltpu.VMEM((2, _CHUNK, _NUM_CHANNELS), jnp.float32),
          pltpu.SemaphoreType.DMA((2,)),
          pltpu.SemaphoreType.DMA((2,)),
      ],
      compiler_params=pltpu.CompilerParams(use_tc_tiling_on_sc=False),
  )
  return f(xf, cls, tab2)


def kernel(x, classes, table):
  tab2 = table.reshape(_NUM_CLASSES, _NUM_CHANNELS, _GRID_DIM).transpose(
      0, 2, 1).reshape(_NUM_CLASSES * _GRID_DIM, _NUM_CHANNELS)
  xf = x.reshape(-1)
  cls = classes.astype(jnp.int32)
  return _run(xf, cls, tab2)
